# baseline (speedup 1.0000x reference)
import functools

import jax
import jax.numpy as jnp
from jax import lax
from jax.experimental import pallas as pl
from jax.experimental.pallas import tpu as pltpu
from jax.experimental.pallas import tpu_sc as plsc

_M0 = 0.05
_NITER = 13
_LO0 = -0.0078125


def _dtm_body(w_ref, dist_ref, o_ref, d2_ref):
    @pl.when(pl.program_id(0) == 0)
    def _():
        d = dist_ref[...]
        d2_ref[...] = d * d

    w = w_ref[0, 0, :][None, :]
    d2 = d2_ref[...]
    bound = _M0 * jnp.sum(w)
    hi = jnp.max(d2, axis=1, keepdims=True)
    lo = jnp.full_like(hi, _LO0)

    def body(_, carry):
        lo_, hi_ = carry
        mid = 0.5 * (lo_ + hi_)
        wm = jnp.sum(jnp.where(d2 <= mid, w, 0.0), axis=1, keepdims=True)
        ge = wm >= bound
        return jnp.where(ge, lo_, mid), jnp.where(ge, mid, hi_)

    lo, hi = jax.lax.fori_loop(0, _NITER, body, (lo, hi))
    big = jnp.float32(3.0e38)
    t2 = jnp.min(jnp.where(d2 > lo, d2, big), axis=1, keepdims=True)
    t2 = jnp.where(t2 >= big, hi, t2)
    below = d2 < t2
    wb = jnp.sum(jnp.where(below, w, 0.0), axis=1, keepdims=True)
    sb = jnp.sum(jnp.where(below, w * d2, 0.0), axis=1, keepdims=True)
    val = sb + t2 * (bound - wb)
    o_ref[...] = jnp.sqrt(val / bound).reshape(o_ref.shape)


def _dtm(weight, dist, hw):
    bc = weight.shape[0]
    return pl.pallas_call(
        _dtm_body,
        grid=(bc,),
        in_specs=[
            pl.BlockSpec((1, 1, hw), lambda i: (i, 0, 0)),
            pl.BlockSpec((hw, hw), lambda i: (0, 0)),
        ],
        out_specs=pl.BlockSpec((1, 1, hw), lambda i: (i, 0, 0)),
        out_shape=jax.ShapeDtypeStruct((bc, 1, hw), jnp.float32),
        scratch_shapes=[pltpu.VMEM((hw, hw), jnp.float32)],
    )(weight.reshape(bc, 1, hw), dist)


_BIG = 3.0e38


def _dtm_sc(weight, dist):
    BC, HW = weight.shape[0], weight.shape[1]
    NW = 32
    RPW = HW // NW
    NCH = HW // 16
    UNR = 8
    mesh = plsc.VectorSubcoreMesh(core_axis_name="c", subcore_axis_name="s")

    _gdn = lax.GatherDimensionNumbers(
        offset_dims=(), collapsed_slice_dims=(0,), start_index_map=(0,))

    def _butterfly(v, op):
        idx = lax.iota(jnp.int32, 16)
        for s in (8, 4, 2, 1):
            sh = lax.gather(v, (idx ^ s)[:, None], _gdn, slice_sizes=(1,),
                            mode=lax.GatherScatterMode.PROMISE_IN_BOUNDS)
            v = op(v, sh)
        return v[0]

    @functools.partial(
        pl.kernel,
        out_type=jax.ShapeDtypeStruct((HW * BC,), jnp.float32),
        mesh=mesh,
        scratch_types=[
            pltpu.VMEM((RPW, HW), jnp.float32),
            pltpu.VMEM((BC, HW), jnp.float32),
            pltpu.VMEM((RPW * BC,), jnp.float32),
            pltpu.SMEM((64,), jnp.float32),
            pltpu.SMEM((RPW,), jnp.float32),
        ],
    )
    def k(w_hbm, dist_hbm, out_hbm, drows, wts, outc, bnds, rmax):
        wid = lax.axis_index("s") * 2 + lax.axis_index("c")
        base = wid * RPW
        pltpu.sync_copy(dist_hbm.at[pl.ds(base, RPW)], drows)
        pltpu.sync_copy(w_hbm, wts)

        zeros16 = jnp.zeros((16,), jnp.float32)

        def bc_bound(bc, _):
            def acc_fn(c, acc):
                return acc + wts[bc, pl.ds(c * 16, 16)]
            acc = lax.fori_loop(0, NCH, acc_fn, zeros16)
            bnds[bc] = _M0 * _butterfly(acc, jnp.add)
            return 0
        lax.fori_loop(0, BC, bc_bound, 0)

        def sq_row(r, _):
            def sq_c(c, mx):
                v = drows[r, pl.ds(c * 16, 16)]
                v2 = v * v
                drows[r, pl.ds(c * 16, 16)] = v2
                return jnp.maximum(mx, v2)
            mx = lax.fori_loop(0, NCH, sq_c, jnp.full((16,), -1.0, jnp.float32))
            rmax[r] = _butterfly(mx, jnp.maximum)
            return 0
        lax.fori_loop(0, RPW, sq_row, 0)

        def row_fn(r, _):
            hi0 = rmax[r]
            lane = lax.iota(jnp.int32, 16)

            def grp_fn(grp, _2):
                def bc_fn(j, vb):
                    bc = grp * 16 + j
                    bound = bnds[bc]

                    def it_fn(it, lh):
                        lo_, hi_ = lh
                        mid = 0.5 * (lo_ + hi_)

                        def acc_fn(g, acc):
                            for u in range(UNR):
                                off = (g * UNR + u) * 16
                                d2v = drows[r, pl.ds(off, 16)]
                                wv = wts[bc, pl.ds(off, 16)]
                                acc = acc + jnp.where(d2v <= mid, wv, 0.0)
                            return acc
                        acc = lax.fori_loop(0, NCH // UNR, acc_fn, zeros16)
                        ge = _butterfly(acc, jnp.add) >= bound
                        return (jnp.where(ge, lo_, mid), jnp.where(ge, mid, hi_))

                    lo, _hi = lax.fori_loop(
                        0, _NITER, it_fn, (jnp.float32(_LO0), hi0))

                    def min_fn(g, mn):
                        for u in range(UNR):
                            off = (g * UNR + u) * 16
                            d2v = drows[r, pl.ds(off, 16)]
                            mn = jnp.minimum(mn, jnp.where(d2v > lo, d2v, _BIG))
                        return mn
                    mn = lax.fori_loop(0, NCH // UNR, min_fn,
                                       jnp.full((16,), _BIG, jnp.float32))
                    t2 = _butterfly(mn, jnp.minimum)
                    t2 = jnp.where(t2 >= _BIG, hi0, t2)

                    def bs_fn(g, wbsb):
                        wb, sb = wbsb
                        for u in range(UNR):
                            off = (g * UNR + u) * 16
                            d2v = drows[r, pl.ds(off, 16)]
                            wv = wts[bc, pl.ds(off, 16)]
                            below = d2v < t2
                            wb = wb + jnp.where(below, wv, 0.0)
                            sb = sb + jnp.where(below, wv * d2v, 0.0)
                        return (wb, sb)
                    wb, sb = lax.fori_loop(
                        0, NCH // UNR, bs_fn, (zeros16, zeros16))
                    val = (_butterfly(sb, jnp.add)
                           + t2 * (bound - _butterfly(wb, jnp.add)))
                    vals, bvec = vb
                    return (jnp.where(lane == j, val, vals),
                            jnp.where(lane == j, bound, bvec))

                vals, bvec = lax.fori_loop(0, 16, bc_fn, (zeros16, zeros16))
                x = vals / bvec
                s = 0.25 + 0.5 * x
                for _n in range(8):
                    s = 0.5 * (s + x / s)
                outc[pl.ds(r * BC + grp * 16, 16)] = s
                return 0
            lax.fori_loop(0, BC // 16, grp_fn, 0)
            return 0
        lax.fori_loop(0, RPW, row_fn, 0)

        pltpu.sync_copy(outc, out_hbm.at[pl.ds(base * BC, RPW * BC)])

    return k(weight, dist)


@jax.jit
def kernel(x, dist):
    B, C, H, W = x.shape
    HW = H * W
    weight = x.reshape(B * C, HW)
    out = _dtm_sc(weight, dist)
    return out.reshape(HW, B * C).T.reshape(B, C, H, W)
